# baseline (speedup 1.0000x reference)
import functools

import jax
import jax.numpy as jnp
from jax import lax
from jax.experimental import pallas as pl
from jax.experimental.pallas import tpu as pltpu
from jax.experimental.pallas import tpu_sc as plsc

N_NODES = 10000
N_EDGES = 320000
D = 128
H = D // 2
DW = D // 2

NC = 2
NS = 16
NW = NC * NS
EPW = N_EDGES // NW
CH = 40
NCH = EPW // CH


def _make_gather_mul():
    mesh = plsc.VectorSubcoreMesh(core_axis_name="c", subcore_axis_name="s")

    @functools.partial(
        pl.kernel,
        out_type=jax.ShapeDtypeStruct((N_EDGES, DW), jnp.float32),
        mesh=mesh,
        compiler_params=pltpu.CompilerParams(
            use_tc_tiling_on_sc=False,
            needs_layout_passes=False,
        ),
        scratch_types=[
            pltpu.VMEM((EPW,), jnp.int32),
            pltpu.VMEM((EPW,), jnp.int32),
            [pltpu.VMEM((CH, DW), jnp.float32) for _ in range(2)],
            [pltpu.VMEM((CH, DW), jnp.float32) for _ in range(2)],
            [pltpu.VMEM((CH, DW), jnp.float32) for _ in range(2)],
            [pltpu.SemaphoreType.DMA for _ in range(2)],
            [pltpu.SemaphoreType.DMA for _ in range(2)],
            [pltpu.SemaphoreType.DMA for _ in range(2)],
        ],
    )
    def gather_mul(src_hbm, dst_hbm, sidx_hbm, didx_hbm, out_hbm,
                   sidx_v, didx_v, srows, drows, orows, sem_s, sem_d, sem_o):
        wid = lax.axis_index("s") * NC + lax.axis_index("c")
        base = wid * EPW
        pltpu.sync_copy(sidx_hbm.at[pl.ds(base, EPW)], sidx_v)
        pltpu.sync_copy(didx_hbm.at[pl.ds(base, EPW)], didx_v)

        def fire_gather(c, b):
            pltpu.async_copy(src_hbm.at[sidx_v.at[pl.ds(c * CH, CH)]],
                             srows[b], sem_s[b])
            pltpu.async_copy(dst_hbm.at[didx_v.at[pl.ds(c * CH, CH)]],
                             drows[b], sem_d[b])

        def wait_gather(b):
            pltpu.make_async_copy(src_hbm.at[sidx_v.at[pl.ds(0, CH)]],
                                  srows[b], sem_s[b]).wait()
            pltpu.make_async_copy(dst_hbm.at[didx_v.at[pl.ds(0, CH)]],
                                  drows[b], sem_d[b]).wait()

        fire_gather(0, 0)
        fire_gather(1, 1)

        def pair_body(k, carry):
            for b in range(2):
                c = 2 * k + b
                wait_gather(b)

                @pl.when(c >= 2)
                def _wait_prev_out():
                    pltpu.make_async_copy(
                        orows[b], out_hbm.at[pl.ds(base, CH)], sem_o[b]).wait()

                def row_body(r, c2):
                    fmt = plsc.PackFormat.INTERLEAVED
                    for j in range(DW // 16):
                        sl = pl.ds(j * 16, 16)
                        s2 = plsc.bitcast(srows[b][r, sl], jnp.bfloat16)
                        d2 = plsc.bitcast(drows[b][r, sl], jnp.bfloat16)
                        s_lo, s_hi = plsc.unpack(s2, format=fmt)
                        d_lo, d_hi = plsc.unpack(d2, format=fmt)
                        p = plsc.pack(s_lo * d_lo, s_hi * d_hi, format=fmt)
                        orows[b][r, sl] = plsc.bitcast(p, jnp.float32)
                    return c2

                lax.fori_loop(0, CH, row_body, 0)
                pltpu.async_copy(orows[b],
                                 out_hbm.at[pl.ds(base + c * CH, CH)],
                                 sem_o[b])

                @pl.when(c + 2 < NCH)
                def _prefetch():
                    fire_gather(c + 2, b)
            return carry

        lax.fori_loop(0, NCH // 2, pair_body, 0)
        for b in range(2):
            pltpu.make_async_copy(
                orows[b], out_hbm.at[pl.ds(base, CH)], sem_o[b]).wait()

    return gather_mul


_gather_mul = _make_gather_mul()

BLK = 4000


def _mlp_body(x_ref, w1_ref, b1_ref, w2_ref, b2_ref, o_ref):
    x = x_ref[...]
    h = jnp.dot(x, w1_ref[...], preferred_element_type=jnp.float32)
    h = jnp.maximum(h + b1_ref[...], 0.0)
    y = jnp.dot(h, w2_ref[...], preferred_element_type=jnp.float32)
    o_ref[...] = jax.nn.sigmoid(y + b2_ref[...])


def _pack_bf16_rows(t):
    b = t.astype(jnp.bfloat16).reshape(t.shape[0], DW, 2)
    return jax.lax.bitcast_convert_type(b, jnp.float32)


def _unpack_bf16_rows(x):
    b = jax.lax.bitcast_convert_type(x, jnp.bfloat16)
    return b.reshape(x.shape[0], D)


def _mlp(x, W1, b1, W2, b2):
    grid = (N_EDGES // BLK,)
    return pl.pallas_call(
        _mlp_body,
        grid=grid,
        in_specs=[
            pl.BlockSpec((BLK, D), lambda i: (i, 0)),
            pl.BlockSpec((D, H), lambda i: (0, 0)),
            pl.BlockSpec((1, H), lambda i: (0, 0)),
            pl.BlockSpec((H, 1), lambda i: (0, 0)),
            pl.BlockSpec((1, 1), lambda i: (0, 0)),
        ],
        out_specs=pl.BlockSpec((BLK, 1), lambda i: (i, 0)),
        out_shape=jax.ShapeDtypeStruct((N_EDGES, 1), jnp.float32),
    )(x, W1, b1, W2, b2)


def kernel(dst_embs, src_embs, edge_indices, W1, b1, W2, b2):
    src_idx = edge_indices[0]
    dst_idx = edge_indices[1]
    xw = _gather_mul(_pack_bf16_rows(src_embs), _pack_bf16_rows(dst_embs),
                     src_idx, dst_idx)
    x = _unpack_bf16_rows(xw)
    return _mlp(x, W1.astype(jnp.bfloat16), b1.reshape(1, H),
                W2, b2.reshape(1, 1))


# speedup vs baseline: 2.8144x; 2.8144x over previous
import functools

import jax
import jax.numpy as jnp
from jax import lax
from jax.experimental import pallas as pl
from jax.experimental.pallas import tpu as pltpu
from jax.experimental.pallas import tpu_sc as plsc

N_NODES = 10000
N_EDGES = 320000
D = 128
H = D // 2

NC = 2
NS = 16
NW = NC * NS
EPW = N_EDGES // NW
CH = 40
NCH = EPW // CH


def _make_gather_mul():
    mesh = plsc.VectorSubcoreMesh(core_axis_name="c", subcore_axis_name="s")

    @functools.partial(
        pl.kernel,
        out_type=jax.ShapeDtypeStruct((N_EDGES, D), jnp.float32),
        mesh=mesh,
        scratch_types=[
            pltpu.VMEM((EPW,), jnp.int32),
            pltpu.VMEM((EPW,), jnp.int32),
            [pltpu.VMEM((CH, D), jnp.float32) for _ in range(2)],
            [pltpu.VMEM((CH, D), jnp.float32) for _ in range(2)],
            [pltpu.VMEM((CH, D), jnp.float32) for _ in range(2)],
            [pltpu.SemaphoreType.DMA for _ in range(2)],
            [pltpu.SemaphoreType.DMA for _ in range(2)],
            [pltpu.SemaphoreType.DMA for _ in range(2)],
        ],
    )
    def gather_mul(src_hbm, dst_hbm, sidx_hbm, didx_hbm, out_hbm,
                   sidx_v, didx_v, srows, drows, orows, sem_s, sem_d, sem_o):
        wid = lax.axis_index("s") * NC + lax.axis_index("c")
        base = wid * EPW
        pltpu.sync_copy(sidx_hbm.at[pl.ds(base, EPW)], sidx_v)
        pltpu.sync_copy(didx_hbm.at[pl.ds(base, EPW)], didx_v)

        def fire_gather(c, b):
            pltpu.async_copy(src_hbm.at[sidx_v.at[pl.ds(c * CH, CH)]],
                             srows[b], sem_s[b])
            pltpu.async_copy(dst_hbm.at[didx_v.at[pl.ds(c * CH, CH)]],
                             drows[b], sem_d[b])

        def wait_gather(b):
            pltpu.make_async_copy(src_hbm.at[sidx_v.at[pl.ds(0, CH)]],
                                  srows[b], sem_s[b]).wait()
            pltpu.make_async_copy(dst_hbm.at[didx_v.at[pl.ds(0, CH)]],
                                  drows[b], sem_d[b]).wait()

        fire_gather(0, 0)
        fire_gather(1, 1)

        def pair_body(k, carry):
            for b in range(2):
                c = 2 * k + b
                wait_gather(b)

                @pl.when(c >= 2)
                def _wait_prev_out():
                    pltpu.make_async_copy(
                        orows[b], out_hbm.at[pl.ds(base, CH)], sem_o[b]).wait()

                @plsc.parallel_loop(0, CH, 1, unroll=4)
                def _row_body(r):
                    for j in range(D // 16):
                        sl = pl.ds(j * 16, 16)
                        orows[b][r, sl] = srows[b][r, sl] * drows[b][r, sl]

                pltpu.async_copy(orows[b],
                                 out_hbm.at[pl.ds(base + c * CH, CH)],
                                 sem_o[b])

                @pl.when(c + 2 < NCH)
                def _prefetch():
                    fire_gather(c + 2, b)
            return carry

        lax.fori_loop(0, NCH // 2, pair_body, 0)
        for b in range(2):
            pltpu.make_async_copy(
                orows[b], out_hbm.at[pl.ds(base, CH)], sem_o[b]).wait()

    return gather_mul


_gather_mul = _make_gather_mul()

BLK = 4000


def _mlp_body(x_ref, w1_ref, b1_ref, w2_ref, b2_ref, o_ref):
    x = x_ref[...]
    h = jnp.dot(x, w1_ref[...], preferred_element_type=jnp.float32)
    h = jnp.maximum(h + b1_ref[...], 0.0)
    y = jnp.dot(h, w2_ref[...], preferred_element_type=jnp.float32)
    o_ref[...] = jax.nn.sigmoid(y + b2_ref[...])


def _mlp(x, W1, b1, W2, b2):
    grid = (N_EDGES // BLK,)
    return pl.pallas_call(
        _mlp_body,
        grid=grid,
        in_specs=[
            pl.BlockSpec((BLK, D), lambda i: (i, 0)),
            pl.BlockSpec((D, H), lambda i: (0, 0)),
            pl.BlockSpec((1, H), lambda i: (0, 0)),
            pl.BlockSpec((H, 1), lambda i: (0, 0)),
            pl.BlockSpec((1, 1), lambda i: (0, 0)),
        ],
        out_specs=pl.BlockSpec((BLK, 1), lambda i: (i, 0)),
        out_shape=jax.ShapeDtypeStruct((N_EDGES, 1), jnp.float32),
    )(x, W1, b1, W2, b2)


def kernel(dst_embs, src_embs, edge_indices, W1, b1, W2, b2):
    src_idx = edge_indices[0]
    dst_idx = edge_indices[1]
    x = _gather_mul(src_embs, dst_embs, src_idx, dst_idx)
    return _mlp(x, W1, b1.reshape(1, H), W2, b2.reshape(1, 1))


# speedup vs baseline: 3.8132x; 1.3549x over previous
import functools

import jax
import jax.numpy as jnp
from jax import lax
from jax.experimental import pallas as pl
from jax.experimental.pallas import tpu as pltpu
from jax.experimental.pallas import tpu_sc as plsc

N_NODES = 10000
N_EDGES = 320000
D = 128
H = D // 2

NC = 2
NS = 16
NW = NC * NS
EPW = N_EDGES // NW
CH = 40
NCH = EPW // CH


def _make_gather_mul():
    mesh = plsc.VectorSubcoreMesh(core_axis_name="c", subcore_axis_name="s")

    @functools.partial(
        pl.kernel,
        out_type=jax.ShapeDtypeStruct((N_EDGES, D), jnp.float32),
        mesh=mesh,
        scratch_types=[
            pltpu.VMEM((EPW,), jnp.int32),
            pltpu.VMEM((EPW,), jnp.int32),
            [pltpu.VMEM((CH, D), jnp.float32) for _ in range(2)],
            [pltpu.VMEM((CH, D), jnp.float32) for _ in range(2)],
            [pltpu.VMEM((CH, D), jnp.float32) for _ in range(2)],
            [pltpu.SemaphoreType.DMA for _ in range(2)],
            [pltpu.SemaphoreType.DMA for _ in range(2)],
            [pltpu.SemaphoreType.DMA for _ in range(2)],
        ],
    )
    def gather_mul(src_hbm, dst_hbm, eidx_hbm, out_hbm,
                   sidx_v, didx_v, srows, drows, orows, sem_s, sem_d, sem_o):
        wid = lax.axis_index("s") * NC + lax.axis_index("c")
        base = wid * EPW
        pltpu.sync_copy(eidx_hbm.at[pl.ds(base, EPW)], sidx_v)
        pltpu.sync_copy(eidx_hbm.at[pl.ds(N_EDGES + base, EPW)], didx_v)

        def fire_gather(c, b):
            pltpu.async_copy(src_hbm.at[sidx_v.at[pl.ds(c * CH, CH)]],
                             srows[b], sem_s[b])
            pltpu.async_copy(dst_hbm.at[didx_v.at[pl.ds(c * CH, CH)]],
                             drows[b], sem_d[b])

        def wait_gather(b):
            pltpu.make_async_copy(src_hbm.at[sidx_v.at[pl.ds(0, CH)]],
                                  srows[b], sem_s[b]).wait()
            pltpu.make_async_copy(dst_hbm.at[didx_v.at[pl.ds(0, CH)]],
                                  drows[b], sem_d[b]).wait()

        fire_gather(0, 0)
        fire_gather(1, 1)

        def pair_body(k, carry):
            for b in range(2):
                c = 2 * k + b
                wait_gather(b)

                @pl.when(c >= 2)
                def _wait_prev_out():
                    pltpu.make_async_copy(
                        orows[b], out_hbm.at[pl.ds(base, CH)], sem_o[b]).wait()

                @plsc.parallel_loop(0, CH, 1, unroll=4)
                def _row_body(r):
                    for j in range(D // 16):
                        sl = pl.ds(j * 16, 16)
                        orows[b][r, sl] = srows[b][r, sl] * drows[b][r, sl]

                pltpu.async_copy(orows[b],
                                 out_hbm.at[pl.ds(base + c * CH, CH)],
                                 sem_o[b])

                @pl.when(c + 2 < NCH)
                def _prefetch():
                    fire_gather(c + 2, b)
            return carry

        lax.fori_loop(0, NCH // 2, pair_body, 0)
        for b in range(2):
            pltpu.make_async_copy(
                orows[b], out_hbm.at[pl.ds(base, CH)], sem_o[b]).wait()

    return gather_mul


_gather_mul = _make_gather_mul()

BLK = 4000
NBLK = N_EDGES // BLK


def _mlp_body(x_ref, w1_ref, b1_ref, w2t_ref, b2_ref, o_ref):
    x = x_ref[...]
    h = jnp.dot(x, w1_ref[...], preferred_element_type=jnp.float32)
    h = jnp.maximum(h + b1_ref[...], 0.0)
    y = jax.lax.dot_general(w2t_ref[...], h, (((1,), (1,)), ((), ())),
                            preferred_element_type=jnp.float32)
    o_ref[pl.ds(pl.program_id(0), 1), :] = jax.nn.sigmoid(y + b2_ref[...])


def _mlp(x, W1, b1, W2t, b2):
    grid = (NBLK,)
    return pl.pallas_call(
        _mlp_body,
        grid=grid,
        in_specs=[
            pl.BlockSpec((BLK, D), lambda i: (i, 0)),
            pl.BlockSpec((D, H), lambda i: (0, 0)),
            pl.BlockSpec((1, H), lambda i: (0, 0)),
            pl.BlockSpec((1, H), lambda i: (0, 0)),
            pl.BlockSpec((1, 1), lambda i: (0, 0)),
        ],
        out_specs=pl.BlockSpec((NBLK, BLK), lambda i: (0, 0)),
        out_shape=jax.ShapeDtypeStruct((NBLK, BLK), jnp.float32),
    )(x, W1, b1, W2t, b2)


def kernel(dst_embs, src_embs, edge_indices, W1, b1, W2, b2):
    x = _gather_mul(src_embs, dst_embs, edge_indices.reshape(2 * N_EDGES))
    y = _mlp(x, W1, b1.reshape(1, H), W2.reshape(1, H), b2.reshape(1, 1))
    return y.reshape(N_EDGES, 1)


# speedup vs baseline: 4.7300x; 1.2404x over previous
import functools

import jax
import jax.numpy as jnp
from jax import lax
from jax.experimental import pallas as pl
from jax.experimental.pallas import tpu as pltpu
from jax.experimental.pallas import tpu_sc as plsc

N_NODES = 10000
N_EDGES = 320000
D = 128
H = D // 2

NC = 2
NS = 16
NW = NC * NS
EPW = N_EDGES // NW
CH = 40
NCH = EPW // CH
PIPE = 5


def _make_gather_mul():
    mesh = plsc.VectorSubcoreMesh(core_axis_name="c", subcore_axis_name="s")

    @functools.partial(
        pl.kernel,
        out_type=jax.ShapeDtypeStruct((N_EDGES, D), jnp.float32),
        mesh=mesh,
        scratch_types=[
            pltpu.VMEM((EPW,), jnp.int32),
            pltpu.VMEM((EPW,), jnp.int32),
            [pltpu.VMEM((CH, D), jnp.float32) for _ in range(PIPE)],
            [pltpu.VMEM((CH, D), jnp.float32) for _ in range(PIPE)],
            [pltpu.VMEM((CH, D), jnp.float32) for _ in range(PIPE)],
            [pltpu.SemaphoreType.DMA for _ in range(PIPE)],
            [pltpu.SemaphoreType.DMA for _ in range(PIPE)],
            [pltpu.SemaphoreType.DMA for _ in range(PIPE)],
        ],
    )
    def gather_mul(src_hbm, dst_hbm, eidx_hbm, out_hbm,
                   sidx_v, didx_v, srows, drows, orows, sem_s, sem_d, sem_o):
        wid = lax.axis_index("s") * NC + lax.axis_index("c")
        base = wid * EPW
        pltpu.sync_copy(eidx_hbm.at[pl.ds(base, EPW)], sidx_v)
        pltpu.sync_copy(eidx_hbm.at[pl.ds(N_EDGES + base, EPW)], didx_v)

        def fire_gather(c, b):
            pltpu.async_copy(src_hbm.at[sidx_v.at[pl.ds(c * CH, CH)]],
                             srows[b], sem_s[b])
            pltpu.async_copy(dst_hbm.at[didx_v.at[pl.ds(c * CH, CH)]],
                             drows[b], sem_d[b])

        def wait_gather(b):
            pltpu.make_async_copy(src_hbm.at[sidx_v.at[pl.ds(0, CH)]],
                                  srows[b], sem_s[b]).wait()
            pltpu.make_async_copy(dst_hbm.at[didx_v.at[pl.ds(0, CH)]],
                                  drows[b], sem_d[b]).wait()

        for p in range(PIPE):
            fire_gather(p, p)

        def round_body(k, carry):
            for b in range(PIPE):
                c = PIPE * k + b
                wait_gather(b)

                @pl.when(c >= PIPE)
                def _wait_prev_out():
                    pltpu.make_async_copy(
                        orows[b], out_hbm.at[pl.ds(base, CH)], sem_o[b]).wait()

                @plsc.parallel_loop(0, CH, 1, unroll=4)
                def _row_body(r):
                    for j in range(D // 16):
                        sl = pl.ds(j * 16, 16)
                        orows[b][r, sl] = srows[b][r, sl] * drows[b][r, sl]

                pltpu.async_copy(orows[b],
                                 out_hbm.at[pl.ds(base + c * CH, CH)],
                                 sem_o[b])

                @pl.when(c + PIPE < NCH)
                def _prefetch():
                    fire_gather(c + PIPE, b)
            return carry

        lax.fori_loop(0, NCH // PIPE, round_body, 0)
        for b in range(PIPE):
            pltpu.make_async_copy(
                orows[b], out_hbm.at[pl.ds(base, CH)], sem_o[b]).wait()

    return gather_mul


_gather_mul = _make_gather_mul()

BLK = 8000
NBLK = N_EDGES // BLK


def _mlp_body(x_ref, w1_ref, b1_ref, w2t_ref, b2_ref, o_ref):
    x = x_ref[...]
    h = jnp.dot(x, w1_ref[...], preferred_element_type=jnp.float32)
    h = jnp.maximum(h + b1_ref[...], 0.0)
    y = jax.lax.dot_general(w2t_ref[...], h, (((1,), (1,)), ((), ())),
                            preferred_element_type=jnp.float32)
    o_ref[pl.ds(pl.program_id(0), 1), :] = jax.nn.sigmoid(y + b2_ref[...])


def _mlp(x, W1, b1, W2t, b2):
    grid = (NBLK,)
    return pl.pallas_call(
        _mlp_body,
        grid=grid,
        in_specs=[
            pl.BlockSpec((BLK, D), lambda i: (i, 0)),
            pl.BlockSpec((D, H), lambda i: (0, 0)),
            pl.BlockSpec((1, H), lambda i: (0, 0)),
            pl.BlockSpec((1, H), lambda i: (0, 0)),
            pl.BlockSpec((1, 1), lambda i: (0, 0)),
        ],
        out_specs=pl.BlockSpec((NBLK, BLK), lambda i: (0, 0)),
        out_shape=jax.ShapeDtypeStruct((NBLK, BLK), jnp.float32),
    )(x, W1, b1, W2t, b2)


def kernel(dst_embs, src_embs, edge_indices, W1, b1, W2, b2):
    x = _gather_mul(src_embs, dst_embs, edge_indices.reshape(2 * N_EDGES))
    y = _mlp(x, W1, b1.reshape(1, H), W2.reshape(1, H), b2.reshape(1, 1))
    return y.reshape(N_EDGES, 1)


# speedup vs baseline: 4.8649x; 1.0285x over previous
import functools

import jax
import jax.numpy as jnp
from jax import lax
from jax.experimental import pallas as pl
from jax.experimental.pallas import tpu as pltpu
from jax.experimental.pallas import tpu_sc as plsc

N_NODES = 10000
N_EDGES = 320000
D = 128
H = D // 2

NC = 2
NS = 16
NW = NC * NS
SLABS = 2
SLAB = N_EDGES // SLABS
EPW = SLAB // NW
CH = 40
NCH = EPW // CH
PIPE = 5


def _make_gather_mul(slab):
    src_off = slab * SLAB
    dst_off = N_EDGES + slab * SLAB
    mesh = plsc.VectorSubcoreMesh(core_axis_name="c", subcore_axis_name="s")

    @functools.partial(
        pl.kernel,
        out_type=jax.ShapeDtypeStruct((SLAB, D), jnp.float32),
        mesh=mesh,
        scratch_types=[
            pltpu.VMEM((EPW,), jnp.int32),
            pltpu.VMEM((EPW,), jnp.int32),
            [pltpu.VMEM((CH, D), jnp.float32) for _ in range(PIPE)],
            [pltpu.VMEM((CH, D), jnp.float32) for _ in range(PIPE)],
            [pltpu.VMEM((CH, D), jnp.float32) for _ in range(PIPE)],
            [pltpu.SemaphoreType.DMA for _ in range(PIPE)],
            [pltpu.SemaphoreType.DMA for _ in range(PIPE)],
            [pltpu.SemaphoreType.DMA for _ in range(PIPE)],
        ],
    )
    def gather_mul(src_hbm, dst_hbm, eidx_hbm, out_hbm,
                   sidx_v, didx_v, srows, drows, orows, sem_s, sem_d, sem_o):
        wid = lax.axis_index("s") * NC + lax.axis_index("c")
        base = wid * EPW
        pltpu.sync_copy(eidx_hbm.at[pl.ds(src_off + base, EPW)], sidx_v)
        pltpu.sync_copy(eidx_hbm.at[pl.ds(dst_off + base, EPW)], didx_v)

        def fire_gather(c, b):
            pltpu.async_copy(src_hbm.at[sidx_v.at[pl.ds(c * CH, CH)]],
                             srows[b], sem_s[b])
            pltpu.async_copy(dst_hbm.at[didx_v.at[pl.ds(c * CH, CH)]],
                             drows[b], sem_d[b])

        def wait_gather(b):
            pltpu.make_async_copy(src_hbm.at[sidx_v.at[pl.ds(0, CH)]],
                                  srows[b], sem_s[b]).wait()
            pltpu.make_async_copy(dst_hbm.at[didx_v.at[pl.ds(0, CH)]],
                                  drows[b], sem_d[b]).wait()

        for p in range(PIPE):
            fire_gather(p, p)

        def round_body(k, carry):
            for b in range(PIPE):
                c = PIPE * k + b
                wait_gather(b)

                @pl.when(c >= PIPE)
                def _wait_prev_out():
                    pltpu.make_async_copy(
                        orows[b], out_hbm.at[pl.ds(base, CH)], sem_o[b]).wait()

                @plsc.parallel_loop(0, CH, 1, unroll=4)
                def _row_body(r):
                    for j in range(D // 16):
                        sl = pl.ds(j * 16, 16)
                        orows[b][r, sl] = srows[b][r, sl] * drows[b][r, sl]

                pltpu.async_copy(orows[b],
                                 out_hbm.at[pl.ds(base + c * CH, CH)],
                                 sem_o[b])

                @pl.when(c + PIPE < NCH)
                def _prefetch():
                    fire_gather(c + PIPE, b)
            return carry

        lax.fori_loop(0, NCH // PIPE, round_body, 0)
        for b in range(PIPE):
            pltpu.make_async_copy(
                orows[b], out_hbm.at[pl.ds(base, CH)], sem_o[b]).wait()

    return gather_mul


_gather_muls = [_make_gather_mul(s) for s in range(SLABS)]

BLK = 8000
NBLK = SLAB // BLK


def _mlp_body(x_ref, w1_ref, b1_ref, w2t_ref, b2_ref, o_ref):
    x = x_ref[...]
    h = jnp.dot(x, w1_ref[...], preferred_element_type=jnp.float32)
    h = jnp.maximum(h + b1_ref[...], 0.0)
    y = jax.lax.dot_general(w2t_ref[...], h, (((1,), (1,)), ((), ())),
                            preferred_element_type=jnp.float32)
    o_ref[pl.ds(pl.program_id(0), 1), :] = jax.nn.sigmoid(y + b2_ref[...])


def _mlp(x, W1, b1, W2t, b2):
    grid = (NBLK,)
    return pl.pallas_call(
        _mlp_body,
        grid=grid,
        in_specs=[
            pl.BlockSpec((BLK, D), lambda i: (i, 0)),
            pl.BlockSpec((D, H), lambda i: (0, 0)),
            pl.BlockSpec((1, H), lambda i: (0, 0)),
            pl.BlockSpec((1, H), lambda i: (0, 0)),
            pl.BlockSpec((1, 1), lambda i: (0, 0)),
        ],
        out_specs=pl.BlockSpec((NBLK, BLK), lambda i: (0, 0)),
        out_shape=jax.ShapeDtypeStruct((NBLK, BLK), jnp.float32),
    )(x, W1, b1, W2t, b2)


def kernel(dst_embs, src_embs, edge_indices, W1, b1, W2, b2):
    eidx = edge_indices.reshape(2 * N_EDGES)
    b1r, w2t, b2r = b1.reshape(1, H), W2.reshape(1, H), b2.reshape(1, 1)
    ys = []
    for s in range(SLABS):
        x = _gather_muls[s](src_embs, dst_embs, eidx)
        ys.append(_mlp(x, W1, b1r, w2t, b2r))
    y = jnp.concatenate(ys, axis=0)
    return y.reshape(N_EDGES, 1)
